# baseline (speedup 1.0000x reference)
import jax
import jax.numpy as jnp
from jax.experimental import pallas as pl
from jax.experimental.pallas import tpu as pltpu

N, T1, T2, K = 512, 64, 64, 3
LOG2E = 1.4426950408889634


def _stblock_kernel(x_ref, a_ref, vs_ref, bs_ref, w1_ref, w2_ref, w3_ref,
                    wc_ref, bc_ref, wconv_ref, bconv_ref, out_ref,
                    am_s, vsb_s, bs2_s, bsmax_s, cc_s, w12_s, ones_s):
    @pl.when(pl.program_id(0) == 0)
    def _init():
        row = jax.lax.broadcasted_iota(jnp.int32, (N, N), 0)
        col = jax.lax.broadcasted_iota(jnp.int32, (N, N), 1)
        am_s[...] = jnp.where(row == col, 0.0, a_ref[...])
        vsb_s[0:N, :] = vs_ref[...].astype(jnp.bfloat16)
        vsb_s[N:N + 8, :] = jnp.ones((8, N), jnp.bfloat16)
        bs2 = bs_ref[...] * LOG2E
        bs2_s[...] = bs2
        bsmax_s[...] = jnp.max(bs2, axis=0, keepdims=True)
        s = jax.lax.broadcasted_iota(jnp.int32, (T2, T2), 0)
        t = jax.lax.broadcasted_iota(jnp.int32, (T2, T2), 1)
        w0, w1c, w2c = wconv_ref[0, 0], wconv_ref[0, 1], wconv_ref[0, 2]
        cc_s[...] = jnp.where(s == t - 1, w0,
                    jnp.where(s == t, w1c,
                    jnp.where(s == t + 1, w2c, 0.0)))
        lane = jax.lax.broadcasted_iota(jnp.int32, (T1, 128), 1)
        w1col = w1_ref[...].reshape(T1, 1)
        w2col = w2_ref[...].reshape(T1, 1)
        w12_s[...] = jnp.where(lane == 0, w1col * LOG2E,
                     jnp.where(lane == 1, w2col, 0.0))
        ones_s[...] = jnp.ones((N, 128), jnp.bfloat16)

    w3 = w3_ref[0, 0]
    wc = wc_ref[...]

    for ib in range(x_ref.shape[0]):
        x = x_ref[ib]

        wv = jnp.dot(x, w12_s[...], preferred_element_type=jnp.float32)
        w1x = wv[:, 0:1]
        v = w3 * wv[:, 1:2]
        vrow = v.reshape(1, N)

        mx = jnp.max(w1x)
        mn = jnp.min(w1x)
        c = jnp.where(vrow >= 0.0, mx * vrow, mn * vrow) + bsmax_s[...]

        e1 = jnp.exp2(w1x * vrow + bs2_s[...] - c).astype(jnp.bfloat16)

        ge = jax.lax.dot_general(vsb_s[...], e1, (((1,), (0,)), ((), ())),
                                 preferred_element_type=jnp.float32)
        g = ge[0:N]
        cinv1 = LOG2E / ge[N:N + 1]

        e2 = jnp.exp2(g * cinv1)
        cinv2 = 1.0 / jnp.sum(e2, axis=0, keepdims=True)

        mb = (am_s[...] * e2 * cinv2).astype(jnp.bfloat16)

        degm = jax.lax.dot_general(mb, ones_s[...],
                                   (((1,), (0,)), ((), ())),
                                   preferred_element_type=jnp.float32)
        deg = degm[:, 0:1]
        dinv = jnp.where(deg > 0, jax.lax.rsqrt(deg), 0.0)

        mt_dot = lambda z: jax.lax.dot_general(
            mb, z.astype(jnp.bfloat16), (((0,), (0,)), ((), ())),
            preferred_element_type=jnp.float32)
        tx1 = -dinv * mt_dot(dinv * x)
        tx2 = -2.0 * dinv * mt_dot(dinv * tx1) - x

        out = jnp.dot(x, wc[0], preferred_element_type=jnp.float32)
        out = out + jnp.dot(tx1, wc[1], preferred_element_type=jnp.float32)
        out = out + jnp.dot(tx2, wc[2], preferred_element_type=jnp.float32)
        out = jnp.maximum(out + bc_ref[0][None, :], 0.0)

        y = jnp.dot(out, cc_s[...], preferred_element_type=jnp.float32)
        y = jnp.maximum(y + bconv_ref[0, 0], 0.0)
        out_ref[ib] = y


def kernel(X, A, Vs, bs, W1, W2, W3, Wcheb, bcheb, wconv, bconv):
    B = X.shape[0]
    PB = 8
    x_hat = X.reshape(B, N, T1)
    w1 = W1.reshape(1, T1)
    w2 = W2.reshape(1, T1)
    w3 = W3.reshape(1, 1)
    bc = bcheb.reshape(1, T2)
    wcv = wconv.reshape(1, K)
    bcv = bconv.reshape(1, 1)

    const = lambda shape: pl.BlockSpec(shape, lambda b: (0,) * len(shape))
    out = pl.pallas_call(
        _stblock_kernel,
        grid=(B // PB,),
        in_specs=[
            pl.BlockSpec((PB, N, T1), lambda b: (b, 0, 0)),
            const((N, N)),
            const((N, N)),
            const((N, N)),
            const((1, T1)),
            const((1, T1)),
            const((1, 1)),
            const((K, T1, T2)),
            const((1, T2)),
            const((1, K)),
            const((1, 1)),
        ],
        out_specs=pl.BlockSpec((PB, N, T2), lambda b: (b, 0, 0)),
        out_shape=jax.ShapeDtypeStruct((B, N, T2), jnp.float32),
        scratch_shapes=[
            pltpu.VMEM((N, N), jnp.float32),
            pltpu.VMEM((N + 8, N), jnp.bfloat16),
            pltpu.VMEM((N, N), jnp.float32),
            pltpu.VMEM((1, N), jnp.float32),
            pltpu.VMEM((T2, T2), jnp.float32),
            pltpu.VMEM((T1, 128), jnp.float32),
            pltpu.VMEM((N, 128), jnp.bfloat16),
        ],
    )(x_hat, A, Vs, bs, w1, w2, w3, Wcheb, bc, wcv, bcv)
    return out.reshape(B, N, 1, T2)


# speedup vs baseline: 1.2892x; 1.2892x over previous
import jax
import jax.numpy as jnp
from jax.experimental import pallas as pl
from jax.experimental.pallas import tpu as pltpu

N, T1, T2, K = 512, 64, 64, 3
LOG2E = 1.4426950408889634


def _stblock_kernel(x_ref, a_ref, vs_ref, bs_ref, w1_ref, w2_ref, w3_ref,
                    wc_ref, bc_ref, wconv_ref, bconv_ref, out_ref,
                    am_s, vsb_s, bs2_s, bsmax_s, cc_s):
    @pl.when(pl.program_id(0) == 0)
    def _init():
        row = jax.lax.broadcasted_iota(jnp.int32, (N, N), 0)
        col = jax.lax.broadcasted_iota(jnp.int32, (N, N), 1)
        am_s[...] = jnp.where(row == col, 0.0, a_ref[...])
        vsb_s[...] = vs_ref[...].astype(jnp.bfloat16)
        bs2 = bs_ref[...] * LOG2E
        bs2_s[...] = bs2
        bsmax_s[...] = jnp.max(bs2, axis=0, keepdims=True)
        s = jax.lax.broadcasted_iota(jnp.int32, (T2, T2), 0)
        t = jax.lax.broadcasted_iota(jnp.int32, (T2, T2), 1)
        w0, w1c, w2c = wconv_ref[0, 0], wconv_ref[0, 1], wconv_ref[0, 2]
        cc_s[...] = jnp.where(s == t - 1, w0,
                    jnp.where(s == t, w1c,
                    jnp.where(s == t + 1, w2c, 0.0)))
    w1 = w1_ref[0]
    w2 = w2_ref[0]
    w3 = w3_ref[0, 0]
    wc = wc_ref[...]

    for ib in range(x_ref.shape[0]):
        x = x_ref[ib]

        w1x = LOG2E * jnp.sum(x * w1[None, :], axis=1, keepdims=True)
        v = w3 * jnp.sum(x * w2[None, :], axis=1, keepdims=True)
        vrow = v.reshape(1, N)

        mx = jnp.max(w1x)
        mn = jnp.min(w1x)
        c = jnp.where(vrow >= 0.0, mx * vrow, mn * vrow) + bsmax_s[...]

        e1 = jnp.exp2(w1x * vrow + bs2_s[...] - c)
        cinv1 = LOG2E / jnp.sum(e1, axis=0, keepdims=True)

        g = jax.lax.dot_general(vsb_s[...], e1.astype(jnp.bfloat16),
                                (((1,), (0,)), ((), ())),
                                preferred_element_type=jnp.float32)

        e2 = jnp.exp2(g * cinv1)
        cinv2 = 1.0 / jnp.sum(e2, axis=0, keepdims=True)

        m = am_s[...] * e2 * cinv2

        deg = jnp.sum(m, axis=1, keepdims=True)
        dinv = jnp.where(deg > 0, jax.lax.rsqrt(deg), 0.0)

        mb = m.astype(jnp.bfloat16)
        mt_dot = lambda z: jax.lax.dot_general(
            mb, z.astype(jnp.bfloat16), (((0,), (0,)), ((), ())),
            preferred_element_type=jnp.float32)
        tx1 = -dinv * mt_dot(dinv * x)
        tx2 = -2.0 * dinv * mt_dot(dinv * tx1) - x

        out = jnp.dot(x, wc[0], preferred_element_type=jnp.float32)
        out = out + jnp.dot(tx1, wc[1], preferred_element_type=jnp.float32)
        out = out + jnp.dot(tx2, wc[2], preferred_element_type=jnp.float32)
        out = jnp.maximum(out + bc_ref[0][None, :], 0.0)

        y = jnp.dot(out, cc_s[...], preferred_element_type=jnp.float32)
        y = jnp.maximum(y + bconv_ref[0, 0], 0.0)
        out_ref[ib] = y


def kernel(X, A, Vs, bs, W1, W2, W3, Wcheb, bcheb, wconv, bconv):
    B = X.shape[0]
    PB = 8
    x_hat = X.reshape(B, N, T1)
    w1 = W1.reshape(1, T1)
    w2 = W2.reshape(1, T1)
    w3 = W3.reshape(1, 1)
    bc = bcheb.reshape(1, T2)
    wcv = wconv.reshape(1, K)
    bcv = bconv.reshape(1, 1)

    const = lambda shape: pl.BlockSpec(shape, lambda b: (0,) * len(shape))
    out = pl.pallas_call(
        _stblock_kernel,
        grid=(B // PB,),
        in_specs=[
            pl.BlockSpec((PB, N, T1), lambda b: (b, 0, 0)),
            const((N, N)),
            const((N, N)),
            const((N, N)),
            const((1, T1)),
            const((1, T1)),
            const((1, 1)),
            const((K, T1, T2)),
            const((1, T2)),
            const((1, K)),
            const((1, 1)),
        ],
        out_specs=pl.BlockSpec((PB, N, T2), lambda b: (b, 0, 0)),
        out_shape=jax.ShapeDtypeStruct((B, N, T2), jnp.float32),
        scratch_shapes=[
            pltpu.VMEM((N, N), jnp.float32),
            pltpu.VMEM((N, N), jnp.bfloat16),
            pltpu.VMEM((N, N), jnp.float32),
            pltpu.VMEM((1, N), jnp.float32),
            pltpu.VMEM((T2, T2), jnp.float32),
        ],
    )(x_hat, A, Vs, bs, w1, w2, w3, Wcheb, bc, wcv, bcv)
    return out.reshape(B, N, 1, T2)
